# baseline (speedup 1.0000x reference)
import functools

import jax
import jax.numpy as jnp
from jax import lax
from jax.experimental import pallas as pl
from jax.experimental.pallas import tpu as pltpu
from jax.experimental.pallas import tpu_sc as plsc

N = 10000
NP = 10240
D = 128
E = 320000
NC, NS = 2, 16
NW = NC * NS
CB = 128
EP = 327680
CH = EP // (NW * CB)
NCHUNK = EP // CB
G = 32
CH0 = 32
CH1 = 128
RPT = NP // NS
BR = 512
GRID = NP // BR

_sc_mesh = plsc.VectorSubcoreMesh(core_axis_name="c", subcore_axis_name="s")


@functools.partial(
    pl.kernel,
    out_type=jax.ShapeDtypeStruct((NC, NP), jnp.float32),
    mesh=_sc_mesh,
    scratch_types=[
        pltpu.VMEM_SHARED((NP,), jnp.float32),
        pltpu.VMEM((CH, 2, CB), jnp.int32),
        pltpu.VMEM((CB,), jnp.float32),
        pltpu.VMEM((RPT,), jnp.float32),
    ],
)
def _sc_hist(sd_hbm, out_hbm, dega, sdb, ones, zb):
    cid = lax.axis_index("c")
    sid = lax.axis_index("s")
    wid = sid * NC + cid
    base = wid * CH
    z16 = jnp.zeros((16,), jnp.float32)
    o16 = jnp.ones((16,), jnp.float32)

    def _zfill(i, _):
        zb[pl.ds(i * 16, 16)] = z16
        return 0

    lax.fori_loop(0, RPT // 16, _zfill, 0)
    for i in range(CB // 16):
        ones[pl.ds(i * 16, 16)] = o16
    rs = pl.ds(sid * RPT, RPT)
    pltpu.sync_copy(zb, dega.at[rs])
    plsc.subcore_barrier()

    pltpu.sync_copy(sd_hbm.at[pl.ds(base, CH)], sdb)

    def _body(j, _):
        pltpu.sync_copy(ones, dega.at[sdb.at[j, 1]], add=True)
        return 0

    lax.fori_loop(0, CH, _body, 0)
    plsc.subcore_barrier()
    pltpu.sync_copy(dega.at[rs], out_hbm.at[cid, rs])


@functools.partial(
    pl.kernel,
    out_type=jax.ShapeDtypeStruct((NC, NP, D), jnp.float32),
    mesh=_sc_mesh,
    scratch_types=[
        pltpu.VMEM_SHARED((NP, D), jnp.float32),
        pltpu.VMEM((G, 2, CB), jnp.int32),
        pltpu.VMEM((CB, D), jnp.float32),
        pltpu.VMEM((CB, D), jnp.float32),
        pltpu.SemaphoreType.DMA,
        pltpu.SemaphoreType.DMA,
    ],
)
def _sc_scatter(tab_hbm, zero_hbm, sd_hbm, out_hbm,
                acc, sdb, rows0, rows1, sem0, sem1):
    cid = lax.axis_index("c")
    sid = lax.axis_index("s")
    rs = pl.ds(sid * RPT, RPT)

    @pl.when(cid == 0)
    def _():
        pltpu.sync_copy(tab_hbm.at[rs], acc.at[rs])

    @pl.when(cid == 1)
    def _():
        pltpu.sync_copy(zero_hbm.at[rs], acc.at[rs])

    plsc.subcore_barrier()

    def _body(i, _):
        j0 = 2 * i
        pltpu.async_copy(tab_hbm.at[sdb.at[j0 + 1, 0]], rows1, sem1)
        pltpu.make_async_copy(tab_hbm.at[sdb.at[j0, 0]], rows0, sem0).wait()
        pltpu.sync_copy(rows0, acc.at[sdb.at[j0, 1]], add=True)

        @pl.when(j0 + 2 < G)
        def _():
            pltpu.async_copy(tab_hbm.at[sdb.at[j0 + 2, 0]], rows0, sem0)

        pltpu.make_async_copy(tab_hbm.at[sdb.at[j0 + 1, 0]], rows1, sem1).wait()
        pltpu.sync_copy(rows1, acc.at[sdb.at[j0 + 1, 1]], add=True)
        return 0

    def _edge_loop(base, nchunks):
        for g in range(nchunks // G):
            pltpu.sync_copy(sd_hbm.at[pl.ds(base + g * G, G)], sdb)
            pltpu.async_copy(tab_hbm.at[sdb.at[0, 0]], rows0, sem0)
            lax.fori_loop(0, G // 2, _body, 0)

    @pl.when(cid == 0)
    def _():
        _edge_loop(sid * CH0, CH0)

    @pl.when(cid == 1)
    def _():
        _edge_loop(NS * CH0 + sid * CH1, CH1)

    plsc.subcore_barrier()
    pltpu.sync_copy(acc.at[rs], out_hbm.at[cid, rs])


def _mm1_body(deg_ref, x_ref, w_ref, hs_ref, d_ref):
    dd = lax.rsqrt(deg_ref[0] + deg_ref[1] + 1.0)
    h = jnp.dot(x_ref[...], w_ref[...], preferred_element_type=jnp.float32)
    hs_ref[...] = h * dd[:, None]
    d_ref[...] = dd


def _mm1(degp, xp, W1):
    return pl.pallas_call(
        _mm1_body,
        grid=(GRID,),
        in_specs=[
            pl.BlockSpec((NC, BR), lambda i: (0, i)),
            pl.BlockSpec((BR, D), lambda i: (i, 0)),
            pl.BlockSpec((D, D), lambda i: (0, 0)),
        ],
        out_specs=[
            pl.BlockSpec((BR, D), lambda i: (i, 0)),
            pl.BlockSpec((BR,), lambda i: (i,)),
        ],
        out_shape=[
            jax.ShapeDtypeStruct((NP, D), jnp.float32),
            jax.ShapeDtypeStruct((NP,), jnp.float32),
        ],
    )(degp, xp, W1)


def _mid_body(acc_ref, d_ref, b_ref, w_ref, hs_ref):
    a = acc_ref[0] + acc_ref[1]
    dd = d_ref[...]
    z = jnp.maximum(a * dd[:, None] + b_ref[...][None, :], 0.0)
    hs_ref[...] = jnp.dot(z, w_ref[...], preferred_element_type=jnp.float32) * dd[:, None]


def _mid(accp, d, b1, W2):
    return pl.pallas_call(
        _mid_body,
        grid=(GRID,),
        in_specs=[
            pl.BlockSpec((NC, BR, D), lambda i: (0, i, 0)),
            pl.BlockSpec((BR,), lambda i: (i,)),
            pl.BlockSpec((D,), lambda i: (0,)),
            pl.BlockSpec((D, D), lambda i: (0, 0)),
        ],
        out_specs=pl.BlockSpec((BR, D), lambda i: (i, 0)),
        out_shape=jax.ShapeDtypeStruct((NP, D), jnp.float32),
    )(accp, d, b1, W2)


def _fin_body(acc_ref, d_ref, b_ref, wc_ref, bc_ref, out_ref):
    a = acc_ref[0] + acc_ref[1]
    dd = d_ref[...]
    h = jnp.maximum(a * dd[:, None] + b_ref[...][None, :], 0.0)
    out_ref[...] = jnp.dot(h, wc_ref[...], preferred_element_type=jnp.float32) + bc_ref[...][None, :]


def _fin(accp, d, b2, Wc, bc):
    ncls = Wc.shape[1]
    return pl.pallas_call(
        _fin_body,
        grid=(GRID,),
        in_specs=[
            pl.BlockSpec((NC, BR, D), lambda i: (0, i, 0)),
            pl.BlockSpec((BR,), lambda i: (i,)),
            pl.BlockSpec((D,), lambda i: (0,)),
            pl.BlockSpec((D, ncls), lambda i: (0, 0)),
            pl.BlockSpec((ncls,), lambda i: (0,)),
        ],
        out_specs=pl.BlockSpec((BR, ncls), lambda i: (i, 0)),
        out_shape=jax.ShapeDtypeStruct((NP, ncls), jnp.float32),
    )(accp, d, b2, Wc, bc)


def kernel(x, edge_index, W1, b1, W2, b2, Wc, bc):
    ei = edge_index.astype(jnp.int32)
    padv = jnp.full((EP - E,), N, jnp.int32)
    src = jnp.concatenate([ei[0], padv]).reshape(NCHUNK, 1, CB)
    dst = jnp.concatenate([ei[1], padv]).reshape(NCHUNK, 1, CB)
    sd = jnp.concatenate([src, dst], axis=1)
    xp = jnp.pad(x, ((0, NP - N), (0, 0)))
    zero = jnp.zeros((NP, D), jnp.float32)

    degp = _sc_hist(sd)
    hs1, d = _mm1(degp, xp, W1)
    acc1 = _sc_scatter(hs1, zero, sd)
    hs2 = _mid(acc1, d, b1, W2)
    acc2 = _sc_scatter(hs2, zero, sd)
    logits = _fin(acc2, d, b2, Wc, bc)
    return logits[:N]


# speedup vs baseline: 1.1292x; 1.1292x over previous
import functools

import jax
import jax.numpy as jnp
from jax import lax
from jax.experimental import pallas as pl
from jax.experimental.pallas import tpu as pltpu
from jax.experimental.pallas import tpu_sc as plsc

N = 10000
NP = 10240
D = 128
E = 320000
NC, NS = 2, 16
NW = NC * NS
CB = 128
EP = 327680
CH = EP // (NW * CB)
NCHUNK = EP // CB
G = 32
CH0 = 128
CH1 = 32
RPT = NP // NS
BR = 512
GRID = NP // BR

_sc_mesh = plsc.VectorSubcoreMesh(core_axis_name="c", subcore_axis_name="s")


@functools.partial(
    pl.kernel,
    out_type=jax.ShapeDtypeStruct((NC, NP), jnp.float32),
    mesh=_sc_mesh,
    scratch_types=[
        pltpu.VMEM_SHARED((NP,), jnp.float32),
        pltpu.VMEM((CH, 2, CB), jnp.int32),
        pltpu.VMEM((CB,), jnp.float32),
        pltpu.VMEM((RPT,), jnp.float32),
    ],
)
def _sc_hist(sd_hbm, out_hbm, dega, sdb, ones, zb):
    cid = lax.axis_index("c")
    sid = lax.axis_index("s")
    wid = sid * NC + cid
    base = wid * CH
    z16 = jnp.zeros((16,), jnp.float32)
    o16 = jnp.ones((16,), jnp.float32)

    def _zfill(i, _):
        zb[pl.ds(i * 16, 16)] = z16
        return 0

    lax.fori_loop(0, RPT // 16, _zfill, 0)
    for i in range(CB // 16):
        ones[pl.ds(i * 16, 16)] = o16
    rs = pl.ds(sid * RPT, RPT)
    pltpu.sync_copy(zb, dega.at[rs])
    plsc.subcore_barrier()

    pltpu.sync_copy(sd_hbm.at[pl.ds(base, CH)], sdb)

    def _body(j, _):
        pltpu.sync_copy(ones, dega.at[sdb.at[j, 1]], add=True)
        return 0

    lax.fori_loop(0, CH, _body, 0)
    plsc.subcore_barrier()
    pltpu.sync_copy(dega.at[rs], out_hbm.at[cid, rs])


@functools.partial(
    pl.kernel,
    out_type=jax.ShapeDtypeStruct((NC, NP, D), jnp.float32),
    mesh=_sc_mesh,
    scratch_types=[
        pltpu.VMEM_SHARED((NP, D), jnp.float32),
        pltpu.VMEM((G, 2, CB), jnp.int32),
        pltpu.VMEM((CB, D), jnp.float32),
        pltpu.VMEM((CB, D), jnp.float32),
        pltpu.SemaphoreType.DMA,
        pltpu.SemaphoreType.DMA,
    ],
)
def _sc_scatter(tab_hbm, zero_hbm, sd_hbm, out_hbm,
                acc, sdb, rows0, rows1, sem0, sem1):
    cid = lax.axis_index("c")
    sid = lax.axis_index("s")
    rs = pl.ds(sid * RPT, RPT)

    @pl.when(cid == 0)
    def _():
        pltpu.sync_copy(tab_hbm.at[rs], acc.at[rs])

    @pl.when(cid == 1)
    def _():
        pltpu.sync_copy(zero_hbm.at[rs], acc.at[rs])

    plsc.subcore_barrier()

    def _body(i, _):
        j0 = 2 * i
        pltpu.async_copy(tab_hbm.at[sdb.at[j0 + 1, 0]], rows1, sem1)
        pltpu.make_async_copy(tab_hbm.at[sdb.at[j0, 0]], rows0, sem0).wait()
        pltpu.sync_copy(rows0, acc.at[sdb.at[j0, 1]], add=True)

        @pl.when(j0 + 2 < G)
        def _():
            pltpu.async_copy(tab_hbm.at[sdb.at[j0 + 2, 0]], rows0, sem0)

        pltpu.make_async_copy(tab_hbm.at[sdb.at[j0 + 1, 0]], rows1, sem1).wait()
        pltpu.sync_copy(rows1, acc.at[sdb.at[j0 + 1, 1]], add=True)
        return 0

    def _edge_loop(base, nchunks):
        for g in range(nchunks // G):
            pltpu.sync_copy(sd_hbm.at[pl.ds(base + g * G, G)], sdb)
            pltpu.async_copy(tab_hbm.at[sdb.at[0, 0]], rows0, sem0)
            lax.fori_loop(0, G // 2, _body, 0)

    @pl.when(cid == 0)
    def _():
        _edge_loop(sid * CH0, CH0)

    @pl.when(cid == 1)
    def _():
        _edge_loop(NS * CH0 + sid * CH1, CH1)

    plsc.subcore_barrier()
    pltpu.sync_copy(acc.at[rs], out_hbm.at[cid, rs])


def _mm1_body(deg_ref, x_ref, w_ref, hs_ref, d_ref):
    dd = lax.rsqrt(deg_ref[0] + deg_ref[1] + 1.0)
    h = jnp.dot(x_ref[...], w_ref[...], preferred_element_type=jnp.float32)
    hs_ref[...] = h * dd[:, None]
    d_ref[...] = dd


def _mm1(degp, xp, W1):
    return pl.pallas_call(
        _mm1_body,
        grid=(GRID,),
        in_specs=[
            pl.BlockSpec((NC, BR), lambda i: (0, i)),
            pl.BlockSpec((BR, D), lambda i: (i, 0)),
            pl.BlockSpec((D, D), lambda i: (0, 0)),
        ],
        out_specs=[
            pl.BlockSpec((BR, D), lambda i: (i, 0)),
            pl.BlockSpec((BR,), lambda i: (i,)),
        ],
        out_shape=[
            jax.ShapeDtypeStruct((NP, D), jnp.float32),
            jax.ShapeDtypeStruct((NP,), jnp.float32),
        ],
    )(degp, xp, W1)


def _mid_body(acc_ref, d_ref, b_ref, w_ref, hs_ref):
    a = acc_ref[0] + acc_ref[1]
    dd = d_ref[...]
    z = jnp.maximum(a * dd[:, None] + b_ref[...][None, :], 0.0)
    hs_ref[...] = jnp.dot(z, w_ref[...], preferred_element_type=jnp.float32) * dd[:, None]


def _mid(accp, d, b1, W2):
    return pl.pallas_call(
        _mid_body,
        grid=(GRID,),
        in_specs=[
            pl.BlockSpec((NC, BR, D), lambda i: (0, i, 0)),
            pl.BlockSpec((BR,), lambda i: (i,)),
            pl.BlockSpec((D,), lambda i: (0,)),
            pl.BlockSpec((D, D), lambda i: (0, 0)),
        ],
        out_specs=pl.BlockSpec((BR, D), lambda i: (i, 0)),
        out_shape=jax.ShapeDtypeStruct((NP, D), jnp.float32),
    )(accp, d, b1, W2)


def _fin_body(acc_ref, d_ref, b_ref, wc_ref, bc_ref, out_ref):
    a = acc_ref[0] + acc_ref[1]
    dd = d_ref[...]
    h = jnp.maximum(a * dd[:, None] + b_ref[...][None, :], 0.0)
    out_ref[...] = jnp.dot(h, wc_ref[...], preferred_element_type=jnp.float32) + bc_ref[...][None, :]


def _fin(accp, d, b2, Wc, bc):
    ncls = Wc.shape[1]
    return pl.pallas_call(
        _fin_body,
        grid=(GRID,),
        in_specs=[
            pl.BlockSpec((NC, BR, D), lambda i: (0, i, 0)),
            pl.BlockSpec((BR,), lambda i: (i,)),
            pl.BlockSpec((D,), lambda i: (0,)),
            pl.BlockSpec((D, ncls), lambda i: (0, 0)),
            pl.BlockSpec((ncls,), lambda i: (0,)),
        ],
        out_specs=pl.BlockSpec((BR, ncls), lambda i: (i, 0)),
        out_shape=jax.ShapeDtypeStruct((NP, ncls), jnp.float32),
    )(accp, d, b2, Wc, bc)


def kernel(x, edge_index, W1, b1, W2, b2, Wc, bc):
    ei = edge_index.astype(jnp.int32)
    padv = jnp.full((EP - E,), N, jnp.int32)
    src = jnp.concatenate([ei[0], padv]).reshape(NCHUNK, 1, CB)
    dst = jnp.concatenate([ei[1], padv]).reshape(NCHUNK, 1, CB)
    sd = jnp.concatenate([src, dst], axis=1)
    xp = jnp.pad(x, ((0, NP - N), (0, 0)))
    zero = jnp.zeros((NP, D), jnp.float32)

    degp = _sc_hist(sd)
    hs1, d = _mm1(degp, xp, W1)
    acc1 = _sc_scatter(hs1, zero, sd)
    hs2 = _mid(acc1, d, b1, W2)
    acc2 = _sc_scatter(hs2, zero, sd)
    logits = _fin(acc2, d, b2, Wc, bc)
    return logits[:N]
